# baseline (speedup 1.0000x reference)
import functools

import jax
import jax.numpy as jnp
import numpy as np
from jax import lax
from jax.experimental import pallas as pl
from jax.experimental.pallas import tpu as pltpu
from jax.experimental.pallas import tpu_sc as plsc

_PATCHES_PER_INDEX = 16
_NUM_RANDOM_INDEXES = 4


def _rotl(x, d):
    return ((x << np.uint32(d)) | (x >> np.uint32(32 - d))).astype(np.uint32)


def _threefry_core(k0, k1, x0, x1):
    x0 = x0.astype(np.uint32).copy()
    x1 = x1.astype(np.uint32).copy()
    ks = [np.uint32(k0), np.uint32(k1),
          np.uint32(np.uint32(k0) ^ np.uint32(k1) ^ np.uint32(0x1BD11BDA))]
    rot = [(13, 15, 26, 6), (17, 29, 16, 24)]
    x0 += ks[0]
    x1 += ks[1]
    for i in range(5):
        for d in rot[i % 2]:
            x0 = x0 + x1
            x1 = _rotl(x1, d)
            x1 = x0 ^ x1
        x0 = x0 + ks[(i + 1) % 3]
        x1 = x1 + ks[(i + 2) % 3] + np.uint32(i + 1)
    return x0, x1


def _random_bits(k, shape):
    n = int(np.prod(shape))
    b1, b2 = _threefry_core(k[0], k[1], np.zeros(n, np.uint32),
                            np.arange(n, dtype=np.uint32))
    return (b1 ^ b2).reshape(shape)


def _split(k, num):
    b1, b2 = _threefry_core(k[0], k[1], np.zeros(num, np.uint32),
                            np.arange(num, dtype=np.uint32))
    return np.stack([b1, b2], axis=1)


def _uniform(k, shape):
    bits = _random_bits(k, shape)
    f = ((bits >> np.uint32(9)) | np.uint32(0x3F800000)).view(np.float32)
    return np.maximum(np.float32(0.0), f - np.float32(1.0))


def _randint(k, shape, minval, maxval):
    ka, kb = _split(k, 2)
    hi = _random_bits(ka, shape)
    lo = _random_bits(kb, shape)
    span = np.uint32(maxval - minval)
    m = np.uint64(65536) % np.uint64(span)
    mult = np.uint32((m * m) % np.uint64(span))
    val = ((hi % span) * mult + (lo % span)) % span
    return np.int32(minval) + val.astype(np.int32)


@functools.lru_cache(maxsize=None)
def _static_plan(B, L):
    len_keep = L - _NUM_RANDOM_INDEXES * _PATCHES_PER_INDEX
    k_noise, k_idx = _split(np.array([0, 1], np.uint32), 2)
    noise = _uniform(k_noise, (B, L))
    indexes = _randint(k_idx, (B, _NUM_RANDOM_INDEXES), 0, 11)
    pos = (indexes[:, :, None] * _PATCHES_PER_INDEX
           + np.arange(_PATCHES_PER_INDEX)[None, None, :]).reshape(B, -1)
    noise[np.arange(B)[:, None], pos] = 2.0
    ids_shuffle = np.argsort(noise, axis=1, kind="stable").astype(np.int32)
    ids_restore = np.argsort(ids_shuffle, axis=1, kind="stable").astype(np.int32)
    ids_keep = ids_shuffle[:, :len_keep]
    mask = np.ones((B, L), np.float32)
    mask[:, :len_keep] = 0.0
    mask = np.take_along_axis(mask, ids_restore, axis=1)
    flat_idx = (np.arange(B, dtype=np.int32)[:, None] * L + ids_keep).reshape(-1)
    return flat_idx.astype(np.int32), ids_restore, mask


@functools.lru_cache(maxsize=None)
def _make_gather(n_rows, D, chunk):
    info = plsc.get_sparse_core_info()
    NC, NS = info.num_cores, info.num_subcores
    NW = NC * NS
    rows_per_w = n_rows // NW
    assert n_rows % NW == 0 and rows_per_w % chunk == 0
    n_chunks = rows_per_w // chunk
    mesh = plsc.VectorSubcoreMesh(core_axis_name="c", subcore_axis_name="s")

    @functools.partial(
        pl.kernel, mesh=mesh,
        out_type=jax.ShapeDtypeStruct((n_rows, D), jnp.float32),
        scratch_types=[
            pltpu.VMEM((chunk,), jnp.int32),
            pltpu.VMEM((chunk, D), jnp.float32),
            pltpu.SemaphoreType.DMA,
        ],
        compiler_params=pltpu.CompilerParams(use_tc_tiling_on_sc=False),
    )
    def gather_rows(table_hbm, idx_hbm, out_hbm, idx_v, rows_v, sem):
        wid = lax.axis_index("s") * NC + lax.axis_index("c")
        base = wid * rows_per_w

        def body(c, carry):
            off = base + c * chunk
            pltpu.sync_copy(idx_hbm.at[pl.ds(off, chunk)], idx_v)
            pltpu.async_copy(table_hbm.at[idx_v], rows_v, sem).wait()
            pltpu.sync_copy(rows_v, out_hbm.at[pl.ds(off, chunk)])
            return carry

        lax.fori_loop(0, n_chunks, body, 0)

    return gather_rows


def kernel(x):
    B, L, D = x.shape
    len_keep = L - _NUM_RANDOM_INDEXES * _PATCHES_PER_INDEX
    flat_idx, ids_restore, mask = _static_plan(B, L)
    table = x.reshape(B * L, D)
    gather_rows = _make_gather(B * len_keep, D, 128)
    out = gather_rows(table, jnp.asarray(flat_idx))
    x_masked = out.reshape(B, len_keep, D)
    return (x_masked, jnp.asarray(mask), jnp.asarray(ids_restore))
